# baseline (speedup 1.0000x reference)
import functools

import jax
import jax.numpy as jnp
from jax import lax
from jax.experimental import pallas as pl
from jax.experimental.pallas import tpu as pltpu
from jax.experimental.pallas import tpu_sc as plsc

B = 4
NQ = 8192
NK = 2048
C = 32
CP = 33
TQ = 512
NW = 32
QPW = B * NQ // NW
L = 16
BIAS = 0.03125
SW = 256
NSL = NK // SW


def _tc_topk_body(k_ref, q_ref, i1_ref, i2_ref, i3_ref, w1_ref, w2_ref, w3_ref):
    ks = k_ref[0] * jnp.float32(-2.0)
    qt = q_ref[0]
    kk = jnp.sum(ks * ks, axis=1, keepdims=True) * jnp.float32(0.25)
    qq = jnp.sum(qt * qt, axis=0, keepdims=True) + jnp.float32(BIAS)
    d2 = jnp.dot(ks, qt, preferred_element_type=jnp.float32) + kk + qq
    ki = lax.bitcast_convert_type(d2, jnp.int32)

    mask = jnp.int32(-8)
    big = jnp.full((SW, TQ), jnp.float32(jnp.inf))
    a1, a2, a3 = big, big, big
    for s in range(NSL):
        x = lax.bitcast_convert_type(
            (ki[s * SW:(s + 1) * SW, :] & mask) | s, jnp.float32)
        hi = jnp.maximum(a1, x)
        a1 = jnp.minimum(a1, x)
        hi2 = jnp.maximum(a2, hi)
        a2 = jnp.minimum(a2, hi)
        a3 = jnp.minimum(a3, hi2)

    row = lax.broadcasted_iota(jnp.int32, (SW, TQ), 0).astype(jnp.float32)
    h, nxt = a1, a2
    keys, rows = [], []
    for _ in range(3):
        m = jnp.min(h, axis=0, keepdims=True)
        l = jnp.min(jnp.where(h == m, row, float(SW)), axis=0, keepdims=True)
        keys.append(lax.bitcast_convert_type(m, jnp.int32))
        rows.append(l)
        hit = row == l
        h = jnp.where(hit, nxt, h)
        nxt = jnp.where(hit, a3, nxt)

    cols = [(k & 7) * SW + l.astype(jnp.int32)
            for k, l in zip(keys, rows)]
    unpack = lambda k: jnp.maximum(
        lax.bitcast_convert_type(k & mask, jnp.float32) - jnp.float32(BIAS),
        1e-10)
    w1 = 1.0 / unpack(keys[0])
    w2 = 1.0 / unpack(keys[1])
    w3 = 1.0 / unpack(keys[2])
    s = w1 + w2 + w3
    i1_ref[0] = cols[0]
    i2_ref[0] = cols[1]
    i3_ref[0] = cols[2]
    w1_ref[0] = w1 / s
    w2_ref[0] = w2 / s
    w3_ref[0] = w3 / s


def _tc_topk(k3, qt3, interpret=False):
    grid = (B, NQ // TQ)
    nblk = B * NQ // TQ
    out = pl.pallas_call(
        _tc_topk_body,
        grid=grid,
        in_specs=[
            pl.BlockSpec((1, NK, 3), lambda b, i: (b, 0, 0)),
            pl.BlockSpec((1, 3, TQ), lambda b, i: (b, 0, i)),
        ],
        out_specs=[pl.BlockSpec((1, 1, TQ), lambda b, i: (b * (NQ // TQ) + i, 0, 0))] * 6,
        out_shape=[
            jax.ShapeDtypeStruct((nblk, 1, TQ), jnp.int32),
            jax.ShapeDtypeStruct((nblk, 1, TQ), jnp.int32),
            jax.ShapeDtypeStruct((nblk, 1, TQ), jnp.int32),
            jax.ShapeDtypeStruct((nblk, 1, TQ), jnp.float32),
            jax.ShapeDtypeStruct((nblk, 1, TQ), jnp.float32),
            jax.ShapeDtypeStruct((nblk, 1, TQ), jnp.float32),
        ],
        compiler_params=pltpu.CompilerParams(
            dimension_semantics=("parallel", "parallel"),
        ),
        interpret=interpret,
    )(k3, qt3)
    return out


def _sc_body(i1, i2, i3, w1, w2, w3, vk, out,
             vk_v, i1_v, i2_v, i3_v, w1_v, w2_v, w3_v, out_v):
    cidx = lax.axis_index("c")
    sidx = lax.axis_index("s")
    wid = sidx * 2 + cidx
    b = wid // (NW // B)
    base = wid * QPW

    pltpu.sync_copy(vk.at[b], vk_v)
    pltpu.sync_copy(i1.at[pl.ds(base, QPW)], i1_v)
    pltpu.sync_copy(i2.at[pl.ds(base, QPW)], i2_v)
    pltpu.sync_copy(i3.at[pl.ds(base, QPW)], i3_v)
    pltpu.sync_copy(w1.at[pl.ds(base, QPW)], w1_v)
    pltpu.sync_copy(w2.at[pl.ds(base, QPW)], w2_v)
    pltpu.sync_copy(w3.at[pl.ds(base, QPW)], w3_v)

    lanes = lax.iota(jnp.int32, L)

    def group(g, carry):
        q0 = g * L
        qb = (q0 + lanes) * C
        ia = i1_v[pl.ds(q0, L)] * C
        ib = i2_v[pl.ds(q0, L)] * C
        ic = i3_v[pl.ds(q0, L)] * C
        wa = w1_v[pl.ds(q0, L)]
        wb = w2_v[pl.ds(q0, L)]
        wc = w3_v[pl.ds(q0, L)]
        for ch in range(C):
            col = (lanes + ch) & (C - 1)
            va = plsc.load_gather(vk_v, [ia + col])
            vb = plsc.load_gather(vk_v, [ib + col])
            vc = plsc.load_gather(vk_v, [ic + col])
            acc = wa * va + wb * vb + wc * vc
            plsc.store_scatter(out_v, [qb + col], acc)
        return carry

    lax.fori_loop(0, QPW // L, group, 0)
    pltpu.sync_copy(out_v, out.at[pl.ds(wid * QPW * C, QPW * C)])


@functools.cache
def _sc_combine_fn():
    return functools.partial(
        pl.kernel,
        mesh=plsc.VectorSubcoreMesh(core_axis_name="c", subcore_axis_name="s"),
        out_type=jax.ShapeDtypeStruct((B * NQ * C,), jnp.float32),
        scratch_types=[
            pltpu.VMEM((NK * C,), jnp.float32),
            pltpu.VMEM((QPW,), jnp.int32),
            pltpu.VMEM((QPW,), jnp.int32),
            pltpu.VMEM((QPW,), jnp.int32),
            pltpu.VMEM((QPW,), jnp.float32),
            pltpu.VMEM((QPW,), jnp.float32),
            pltpu.VMEM((QPW,), jnp.float32),
            pltpu.VMEM((QPW * C,), jnp.float32),
        ],
        compiler_params=pltpu.CompilerParams(needs_layout_passes=False),
    )(_sc_body)


@jax.jit
def kernel(xyz_q, xyz_k, v_k):
    qt3 = jnp.swapaxes(xyz_q, 1, 2)
    i1, i2, i3, w1, w2, w3 = _tc_topk(xyz_k, qt3)
    flat = lambda x: x.reshape(B * NQ)
    out = _sc_combine_fn()(flat(i1), flat(i2), flat(i3),
                           flat(w1), flat(w2), flat(w3),
                           v_k.reshape(B, NK * C))
    return out.reshape(B, NQ, C)


# speedup vs baseline: 1.1205x; 1.1205x over previous
import functools

import jax
import jax.numpy as jnp
from jax import lax
from jax.experimental import pallas as pl
from jax.experimental.pallas import tpu as pltpu
from jax.experimental.pallas import tpu_sc as plsc

B = 4
NQ = 8192
NK = 2048
C = 32
CP = 33
TQ = 1024
NW = 32
QPW = B * NQ // NW
L = 16
BIAS = 0.03125
SW = 256
NSL = NK // SW


def _tc_topk_body(k_ref, q_ref, i1_ref, i2_ref, i3_ref, w1_ref, w2_ref, w3_ref):
    ks = k_ref[0] * jnp.float32(-2.0)
    qt = q_ref[0]
    kk = jnp.sum(ks * ks, axis=1, keepdims=True) * jnp.float32(0.25)
    qq = jnp.sum(qt * qt, axis=0, keepdims=True) + jnp.float32(BIAS)
    d2 = jnp.dot(ks, qt, preferred_element_type=jnp.float32) + kk + qq
    ki = lax.bitcast_convert_type(d2, jnp.int32)

    mask = jnp.int32(-8)
    big = jnp.full((SW, TQ), jnp.float32(jnp.inf))
    a1, a2, a3 = big, big, big
    for s in range(NSL):
        x = lax.bitcast_convert_type(
            (ki[s * SW:(s + 1) * SW, :] & mask) | s, jnp.float32)
        hi = jnp.maximum(a1, x)
        a1 = jnp.minimum(a1, x)
        hi2 = jnp.maximum(a2, hi)
        a2 = jnp.minimum(a2, hi)
        a3 = jnp.minimum(a3, hi2)

    row = lax.broadcasted_iota(jnp.int32, (SW, TQ), 0).astype(jnp.float32)
    h, nxt = a1, a2
    keys, rows = [], []
    for _ in range(3):
        m = jnp.min(h, axis=0, keepdims=True)
        l = jnp.min(jnp.where(h == m, row, float(SW)), axis=0, keepdims=True)
        keys.append(lax.bitcast_convert_type(m, jnp.int32))
        rows.append(l)
        hit = row == l
        h = jnp.where(hit, nxt, h)
        nxt = jnp.where(hit, a3, nxt)

    cols = [(k & 7) * SW + l.astype(jnp.int32)
            for k, l in zip(keys, rows)]
    unpack = lambda k: jnp.maximum(
        lax.bitcast_convert_type(k & mask, jnp.float32) - jnp.float32(BIAS),
        1e-10)
    w1 = 1.0 / unpack(keys[0])
    w2 = 1.0 / unpack(keys[1])
    w3 = 1.0 / unpack(keys[2])
    s = w1 + w2 + w3
    i1_ref[0] = cols[0]
    i2_ref[0] = cols[1]
    i3_ref[0] = cols[2]
    w1_ref[0] = w1 / s
    w2_ref[0] = w2 / s
    w3_ref[0] = w3 / s


def _tc_topk(k3, qt3, interpret=False):
    grid = (B, NQ // TQ)
    nblk = B * NQ // TQ
    out = pl.pallas_call(
        _tc_topk_body,
        grid=grid,
        in_specs=[
            pl.BlockSpec((1, NK, 3), lambda b, i: (b, 0, 0)),
            pl.BlockSpec((1, 3, TQ), lambda b, i: (b, 0, i)),
        ],
        out_specs=[pl.BlockSpec((1, 1, TQ), lambda b, i: (b * (NQ // TQ) + i, 0, 0))] * 6,
        out_shape=[
            jax.ShapeDtypeStruct((nblk, 1, TQ), jnp.int32),
            jax.ShapeDtypeStruct((nblk, 1, TQ), jnp.int32),
            jax.ShapeDtypeStruct((nblk, 1, TQ), jnp.int32),
            jax.ShapeDtypeStruct((nblk, 1, TQ), jnp.float32),
            jax.ShapeDtypeStruct((nblk, 1, TQ), jnp.float32),
            jax.ShapeDtypeStruct((nblk, 1, TQ), jnp.float32),
        ],
        compiler_params=pltpu.CompilerParams(
            dimension_semantics=("parallel", "parallel"),
        ),
        interpret=interpret,
    )(k3, qt3)
    return out


def _sc_body(i1, i2, i3, w1, w2, w3, vk, out,
             vk_v, i1_v, i2_v, i3_v, w1_v, w2_v, w3_v, out_v):
    cidx = lax.axis_index("c")
    sidx = lax.axis_index("s")
    wid = sidx * 2 + cidx
    b = wid // (NW // B)
    base = wid * QPW

    pltpu.sync_copy(vk.at[b], vk_v)
    pltpu.sync_copy(i1.at[pl.ds(base, QPW)], i1_v)
    pltpu.sync_copy(i2.at[pl.ds(base, QPW)], i2_v)
    pltpu.sync_copy(i3.at[pl.ds(base, QPW)], i3_v)
    pltpu.sync_copy(w1.at[pl.ds(base, QPW)], w1_v)
    pltpu.sync_copy(w2.at[pl.ds(base, QPW)], w2_v)
    pltpu.sync_copy(w3.at[pl.ds(base, QPW)], w3_v)

    def group(g, carry):
        q0 = g * L
        ia = i1_v[pl.ds(q0, L)] * CP
        ib = i2_v[pl.ds(q0, L)] * CP
        ic = i3_v[pl.ds(q0, L)] * CP
        wa = w1_v[pl.ds(q0, L)]
        wb = w2_v[pl.ds(q0, L)]
        wc = w3_v[pl.ds(q0, L)]
        for ch in range(C):
            va = plsc.load_gather(vk_v, [ia + ch])
            vb = plsc.load_gather(vk_v, [ib + ch])
            vc = plsc.load_gather(vk_v, [ic + ch])
            acc = wa * va + wb * vb + wc * vc
            out_v[pl.ds(ch * QPW + q0, L)] = acc
        return carry

    lax.fori_loop(0, QPW // L, group, 0)
    pltpu.sync_copy(out_v, out.at[pl.ds(wid * QPW * C, QPW * C)])


@functools.cache
def _sc_combine_fn():
    return functools.partial(
        pl.kernel,
        mesh=plsc.VectorSubcoreMesh(core_axis_name="c", subcore_axis_name="s"),
        out_type=jax.ShapeDtypeStruct((NW * C * QPW,), jnp.float32),
        scratch_types=[
            pltpu.VMEM((NK * CP,), jnp.float32),
            pltpu.VMEM((QPW,), jnp.int32),
            pltpu.VMEM((QPW,), jnp.int32),
            pltpu.VMEM((QPW,), jnp.int32),
            pltpu.VMEM((QPW,), jnp.float32),
            pltpu.VMEM((QPW,), jnp.float32),
            pltpu.VMEM((QPW,), jnp.float32),
            pltpu.VMEM((QPW * C,), jnp.float32),
        ],
        compiler_params=pltpu.CompilerParams(needs_layout_passes=False),
    )(_sc_body)


@jax.jit
def kernel(xyz_q, xyz_k, v_k):
    qt3 = jnp.swapaxes(xyz_q, 1, 2)
    i1, i2, i3, w1, w2, w3 = _tc_topk(xyz_k, qt3)
    vkp = jnp.concatenate(
        [v_k, jnp.zeros((B, NK, CP - C), jnp.float32)], axis=-1)
    flat = lambda x: x.reshape(B * NQ)
    out = _sc_combine_fn()(flat(i1), flat(i2), flat(i3),
                           flat(w1), flat(w2), flat(w3),
                           vkp.reshape(B, NK * CP))
    out = out.reshape(B, NW // B, C, QPW).transpose(0, 1, 3, 2)
    return out.reshape(B, NQ, C)


# speedup vs baseline: 1.1602x; 1.0354x over previous
import functools

import jax
import jax.numpy as jnp
from jax import lax
from jax.experimental import pallas as pl
from jax.experimental.pallas import tpu as pltpu
from jax.experimental.pallas import tpu_sc as plsc

B = 4
NQ = 8192
NK = 2048
C = 32
CP = 33
TQ = 2048
NW = 32
QPW = B * NQ // NW
L = 16
BIAS = 0.03125
SW = 256
NSL = NK // SW


def _tc_topk_body(k_ref, q_ref, i1_ref, i2_ref, i3_ref, w1_ref, w2_ref, w3_ref):
    ks = k_ref[0] * jnp.float32(-2.0)
    qt = q_ref[0]
    kk = jnp.sum(ks * ks, axis=1, keepdims=True) * jnp.float32(0.25)
    qq = jnp.sum(qt * qt, axis=0, keepdims=True) + jnp.float32(BIAS)
    d2 = jnp.dot(ks, qt, preferred_element_type=jnp.float32) + kk + qq
    ki = lax.bitcast_convert_type(d2, jnp.int32)

    mask = jnp.int32(-8)
    big = jnp.full((SW, TQ), jnp.float32(jnp.inf))
    a1, a2, a3 = big, big, big
    for s in range(NSL):
        x = lax.bitcast_convert_type(
            (ki[s * SW:(s + 1) * SW, :] & mask) | s, jnp.float32)
        hi = jnp.maximum(a1, x)
        a1 = jnp.minimum(a1, x)
        hi2 = jnp.maximum(a2, hi)
        a2 = jnp.minimum(a2, hi)
        a3 = jnp.minimum(a3, hi2)

    row = lax.broadcasted_iota(jnp.int32, (SW, TQ), 0).astype(jnp.float32)
    h, nxt = a1, a2
    keys, rows = [], []
    for _ in range(3):
        m = jnp.min(h, axis=0, keepdims=True)
        l = jnp.min(jnp.where(h == m, row, float(SW)), axis=0, keepdims=True)
        keys.append(lax.bitcast_convert_type(m, jnp.int32))
        rows.append(l)
        hit = row == l
        h = jnp.where(hit, nxt, h)
        nxt = jnp.where(hit, a3, nxt)

    cols = [(k & 7) * SW + l.astype(jnp.int32)
            for k, l in zip(keys, rows)]
    unpack = lambda k: jnp.maximum(
        lax.bitcast_convert_type(k & mask, jnp.float32) - jnp.float32(BIAS),
        1e-10)
    w1 = 1.0 / unpack(keys[0])
    w2 = 1.0 / unpack(keys[1])
    w3 = 1.0 / unpack(keys[2])
    s = w1 + w2 + w3
    i1_ref[0] = cols[0]
    i2_ref[0] = cols[1]
    i3_ref[0] = cols[2]
    w1_ref[0] = w1 / s
    w2_ref[0] = w2 / s
    w3_ref[0] = w3 / s


def _tc_topk(k3, qt3, interpret=False):
    grid = (B, NQ // TQ)
    nblk = B * NQ // TQ
    out = pl.pallas_call(
        _tc_topk_body,
        grid=grid,
        in_specs=[
            pl.BlockSpec((1, NK, 3), lambda b, i: (b, 0, 0)),
            pl.BlockSpec((1, 3, TQ), lambda b, i: (b, 0, i)),
        ],
        out_specs=[pl.BlockSpec((1, 1, TQ), lambda b, i: (b * (NQ // TQ) + i, 0, 0))] * 6,
        out_shape=[
            jax.ShapeDtypeStruct((nblk, 1, TQ), jnp.int32),
            jax.ShapeDtypeStruct((nblk, 1, TQ), jnp.int32),
            jax.ShapeDtypeStruct((nblk, 1, TQ), jnp.int32),
            jax.ShapeDtypeStruct((nblk, 1, TQ), jnp.float32),
            jax.ShapeDtypeStruct((nblk, 1, TQ), jnp.float32),
            jax.ShapeDtypeStruct((nblk, 1, TQ), jnp.float32),
        ],
        compiler_params=pltpu.CompilerParams(
            dimension_semantics=("parallel", "parallel"),
        ),
        interpret=interpret,
    )(k3, qt3)
    return out


def _sc_body(i1, i2, i3, w1, w2, w3, vk, out,
             vk_v, i1_v, i2_v, i3_v, w1_v, w2_v, w3_v, out_v):
    cidx = lax.axis_index("c")
    sidx = lax.axis_index("s")
    wid = sidx * 2 + cidx
    b = wid // (NW // B)
    base = wid * QPW

    pltpu.sync_copy(vk.at[b], vk_v)
    pltpu.sync_copy(i1.at[pl.ds(base, QPW)], i1_v)
    pltpu.sync_copy(i2.at[pl.ds(base, QPW)], i2_v)
    pltpu.sync_copy(i3.at[pl.ds(base, QPW)], i3_v)
    pltpu.sync_copy(w1.at[pl.ds(base, QPW)], w1_v)
    pltpu.sync_copy(w2.at[pl.ds(base, QPW)], w2_v)
    pltpu.sync_copy(w3.at[pl.ds(base, QPW)], w3_v)

    def group(g, carry):
        q0 = g * L
        ia = i1_v[pl.ds(q0, L)] * CP
        ib = i2_v[pl.ds(q0, L)] * CP
        ic = i3_v[pl.ds(q0, L)] * CP
        wa = w1_v[pl.ds(q0, L)]
        wb = w2_v[pl.ds(q0, L)]
        wc = w3_v[pl.ds(q0, L)]
        for ch in range(C):
            va = plsc.load_gather(vk_v, [ia + ch])
            vb = plsc.load_gather(vk_v, [ib + ch])
            vc = plsc.load_gather(vk_v, [ic + ch])
            acc = wa * va + wb * vb + wc * vc
            out_v[pl.ds(ch * QPW + q0, L)] = acc
        return carry

    lax.fori_loop(0, QPW // L, group, 0)
    pltpu.sync_copy(out_v, out.at[pl.ds(wid * QPW * C, QPW * C)])


@functools.cache
def _sc_combine_fn():
    return functools.partial(
        pl.kernel,
        mesh=plsc.VectorSubcoreMesh(core_axis_name="c", subcore_axis_name="s"),
        out_type=jax.ShapeDtypeStruct((NW * C * QPW,), jnp.float32),
        scratch_types=[
            pltpu.VMEM((NK * CP,), jnp.float32),
            pltpu.VMEM((QPW,), jnp.int32),
            pltpu.VMEM((QPW,), jnp.int32),
            pltpu.VMEM((QPW,), jnp.int32),
            pltpu.VMEM((QPW,), jnp.float32),
            pltpu.VMEM((QPW,), jnp.float32),
            pltpu.VMEM((QPW,), jnp.float32),
            pltpu.VMEM((QPW * C,), jnp.float32),
        ],
        compiler_params=pltpu.CompilerParams(needs_layout_passes=False),
    )(_sc_body)


@jax.jit
def kernel(xyz_q, xyz_k, v_k):
    qt3 = jnp.swapaxes(xyz_q, 1, 2)
    i1, i2, i3, w1, w2, w3 = _tc_topk(xyz_k, qt3)
    vkp = jnp.concatenate(
        [v_k, jnp.zeros((B, NK, CP - C), jnp.float32)], axis=-1)
    flat = lambda x: x.reshape(B * NQ)
    out = _sc_combine_fn()(flat(i1), flat(i2), flat(i3),
                           flat(w1), flat(w2), flat(w3),
                           vkp.reshape(B, NK * CP))
    out = out.reshape(B, NW // B, C, QPW).transpose(0, 1, 3, 2)
    return out.reshape(B, NQ, C)


# speedup vs baseline: 1.1788x; 1.0160x over previous
import functools

import jax
import jax.numpy as jnp
from jax import lax
from jax.experimental import pallas as pl
from jax.experimental.pallas import tpu as pltpu
from jax.experimental.pallas import tpu_sc as plsc

B = 4
NQ = 8192
NK = 2048
C = 32
CP = 33
TQ = 4096
NW = 32
QPW = B * NQ // NW
L = 16
BIAS = 0.03125
SW = 256
NSL = NK // SW


def _tc_topk_body(k_ref, q_ref, i1_ref, i2_ref, i3_ref, w1_ref, w2_ref, w3_ref):
    ks = k_ref[0] * jnp.float32(-2.0)
    qt = q_ref[0]
    kk = jnp.sum(ks * ks, axis=1, keepdims=True) * jnp.float32(0.25)
    qq = jnp.sum(qt * qt, axis=0, keepdims=True) + jnp.float32(BIAS)
    d2 = jnp.dot(ks, qt, preferred_element_type=jnp.float32) + kk + qq
    ki = lax.bitcast_convert_type(d2, jnp.int32)

    mask = jnp.int32(-8)
    big = jnp.full((SW, TQ), jnp.float32(jnp.inf))
    a1, a2, a3 = big, big, big
    for s in range(NSL):
        x = lax.bitcast_convert_type(
            (ki[s * SW:(s + 1) * SW, :] & mask) | s, jnp.float32)
        hi = jnp.maximum(a1, x)
        a1 = jnp.minimum(a1, x)
        hi2 = jnp.maximum(a2, hi)
        a2 = jnp.minimum(a2, hi)
        a3 = jnp.minimum(a3, hi2)

    row = lax.broadcasted_iota(jnp.int32, (SW, TQ), 0).astype(jnp.float32)
    h, nxt = a1, a2
    keys, rows = [], []
    for _ in range(3):
        m = jnp.min(h, axis=0, keepdims=True)
        l = jnp.min(jnp.where(h == m, row, float(SW)), axis=0, keepdims=True)
        keys.append(lax.bitcast_convert_type(m, jnp.int32))
        rows.append(l)
        hit = row == l
        h = jnp.where(hit, nxt, h)
        nxt = jnp.where(hit, a3, nxt)

    cols = [(k & 7) * SW + l.astype(jnp.int32)
            for k, l in zip(keys, rows)]
    unpack = lambda k: jnp.maximum(
        lax.bitcast_convert_type(k & mask, jnp.float32) - jnp.float32(BIAS),
        1e-10)
    w1 = 1.0 / unpack(keys[0])
    w2 = 1.0 / unpack(keys[1])
    w3 = 1.0 / unpack(keys[2])
    s = w1 + w2 + w3
    i1_ref[0] = cols[0]
    i2_ref[0] = cols[1]
    i3_ref[0] = cols[2]
    w1_ref[0] = w1 / s
    w2_ref[0] = w2 / s
    w3_ref[0] = w3 / s


def _tc_topk(k3, qt3, interpret=False):
    grid = (B, NQ // TQ)
    nblk = B * NQ // TQ
    out = pl.pallas_call(
        _tc_topk_body,
        grid=grid,
        in_specs=[
            pl.BlockSpec((1, NK, 3), lambda b, i: (b, 0, 0)),
            pl.BlockSpec((1, 3, TQ), lambda b, i: (b, 0, i)),
        ],
        out_specs=[pl.BlockSpec((1, 1, TQ), lambda b, i: (b * (NQ // TQ) + i, 0, 0))] * 6,
        out_shape=[
            jax.ShapeDtypeStruct((nblk, 1, TQ), jnp.int32),
            jax.ShapeDtypeStruct((nblk, 1, TQ), jnp.int32),
            jax.ShapeDtypeStruct((nblk, 1, TQ), jnp.int32),
            jax.ShapeDtypeStruct((nblk, 1, TQ), jnp.float32),
            jax.ShapeDtypeStruct((nblk, 1, TQ), jnp.float32),
            jax.ShapeDtypeStruct((nblk, 1, TQ), jnp.float32),
        ],
        compiler_params=pltpu.CompilerParams(
            dimension_semantics=("parallel", "parallel"),
        ),
        interpret=interpret,
    )(k3, qt3)
    return out


def _sc_body(i1, i2, i3, w1, w2, w3, vk, out,
             vk_v, i1_v, i2_v, i3_v, w1_v, w2_v, w3_v, out_v):
    cidx = lax.axis_index("c")
    sidx = lax.axis_index("s")
    wid = sidx * 2 + cidx
    b = wid // (NW // B)
    base = wid * QPW

    pltpu.sync_copy(vk.at[b], vk_v)
    pltpu.sync_copy(i1.at[pl.ds(base, QPW)], i1_v)
    pltpu.sync_copy(i2.at[pl.ds(base, QPW)], i2_v)
    pltpu.sync_copy(i3.at[pl.ds(base, QPW)], i3_v)
    pltpu.sync_copy(w1.at[pl.ds(base, QPW)], w1_v)
    pltpu.sync_copy(w2.at[pl.ds(base, QPW)], w2_v)
    pltpu.sync_copy(w3.at[pl.ds(base, QPW)], w3_v)

    def group(g, carry):
        q0 = g * L
        ia = i1_v[pl.ds(q0, L)] * CP
        ib = i2_v[pl.ds(q0, L)] * CP
        ic = i3_v[pl.ds(q0, L)] * CP
        wa = w1_v[pl.ds(q0, L)]
        wb = w2_v[pl.ds(q0, L)]
        wc = w3_v[pl.ds(q0, L)]
        for ch in range(C):
            va = plsc.load_gather(vk_v, [ia + ch])
            vb = plsc.load_gather(vk_v, [ib + ch])
            vc = plsc.load_gather(vk_v, [ic + ch])
            acc = wa * va + wb * vb + wc * vc
            out_v[pl.ds(ch * QPW + q0, L)] = acc
        return carry

    lax.fori_loop(0, QPW // L, group, 0)
    pltpu.sync_copy(out_v, out.at[pl.ds(wid * QPW * C, QPW * C)])


@functools.cache
def _sc_combine_fn():
    return functools.partial(
        pl.kernel,
        mesh=plsc.VectorSubcoreMesh(core_axis_name="c", subcore_axis_name="s"),
        out_type=jax.ShapeDtypeStruct((NW * C * QPW,), jnp.float32),
        scratch_types=[
            pltpu.VMEM((NK * CP,), jnp.float32),
            pltpu.VMEM((QPW,), jnp.int32),
            pltpu.VMEM((QPW,), jnp.int32),
            pltpu.VMEM((QPW,), jnp.int32),
            pltpu.VMEM((QPW,), jnp.float32),
            pltpu.VMEM((QPW,), jnp.float32),
            pltpu.VMEM((QPW,), jnp.float32),
            pltpu.VMEM((QPW * C,), jnp.float32),
        ],
        compiler_params=pltpu.CompilerParams(needs_layout_passes=False),
    )(_sc_body)


@jax.jit
def kernel(xyz_q, xyz_k, v_k):
    qt3 = jnp.swapaxes(xyz_q, 1, 2)
    i1, i2, i3, w1, w2, w3 = _tc_topk(xyz_k, qt3)
    vkp = jnp.concatenate(
        [v_k, jnp.zeros((B, NK, CP - C), jnp.float32)], axis=-1)
    flat = lambda x: x.reshape(B * NQ)
    out = _sc_combine_fn()(flat(i1), flat(i2), flat(i3),
                           flat(w1), flat(w2), flat(w3),
                           vkp.reshape(B, NK * CP))
    out = out.reshape(B, NW // B, C, QPW).transpose(0, 1, 3, 2)
    return out.reshape(B, NQ, C)
